# Initial kernel scaffold; baseline (speedup 1.0000x reference)
import jax
import jax.numpy as jnp
from jax.experimental import pallas as pl


def kernel(roi_cls_loc, roi_scores, rois):
    raise NotImplementedError("write your pallas kernel here")


# speedup vs baseline: 5.9042x; 5.9042x over previous
import functools

import jax
import jax.numpy as jnp
from jax import lax
from jax.experimental import pallas as pl
from jax.experimental.pallas import tpu as pltpu, tpu_sc as plsc

N = 5000
NPAD = 5120
NCHUNK = NPAD // 16
NCLS = 21
NFG = NCLS - 1
K = 100
KPAD = 128
NMS_THRESH = 0.3
SCORE_THRESH = 0.05
IMG_H = 600.0
IMG_W = 800.0
NEG = -1.0


def _prep_body(scores_ref, loc_ref, rois_ref, probs_ref, boxes_ref):
    scores = scores_ref[...]
    mx = jnp.max(scores, axis=0, keepdims=True)
    e = jnp.exp(scores - mx)
    denom = jnp.sum(e, axis=0, keepdims=True)
    col = lax.broadcasted_iota(jnp.int32, (1, NPAD), 1)
    row_valid = col < N

    src_h = rois_ref[2:3, :] - rois_ref[0:1, :]
    src_w = rois_ref[3:4, :] - rois_ref[1:2, :]
    ctr_y = rois_ref[0:1, :] + 0.5 * src_h
    ctr_x = rois_ref[1:2, :] + 0.5 * src_w

    for c in range(1, NCLS):
        dy = loc_ref[c, 0:1, :] * 0.1
        dx = loc_ref[c, 1:2, :] * 0.1
        dh = loc_ref[c, 2:3, :] * 0.2
        dw = loc_ref[c, 3:4, :] * 0.2
        h = jnp.exp(dh) * src_h
        w = jnp.exp(dw) * src_w
        cy = dy * src_h + ctr_y
        cx = dx * src_w + ctr_x
        y1 = jnp.clip(cy - 0.5 * h, 0.0, IMG_H)
        x1 = jnp.clip(cx - 0.5 * w, 0.0, IMG_W)
        y2 = jnp.clip(cy + 0.5 * h, 0.0, IMG_H)
        x2 = jnp.clip(cx + 0.5 * w, 0.0, IMG_W)
        boxes_ref[c - 1, 0:1, :] = y1
        boxes_ref[c - 1, 1:2, :] = x1
        boxes_ref[c - 1, 2:3, :] = y2
        boxes_ref[c - 1, 3:4, :] = x2
        prob = e[c : c + 1, :] / denom
        probs_ref[c - 1 : c, :] = jnp.where(row_valid, prob, 0.0)


def _nms_body(probs_hbm, boxes_hbm, ob_hbm, os_hbm, ol_hbm,
              score_v, boxes_v, a2_v, ob_v, os_v, ol_v):
    cidx = lax.axis_index("c")
    sidx = lax.axis_index("s")
    wid = sidx * 2 + cidx

    @pl.when(wid < NFG)
    def _work():
        pltpu.sync_copy(probs_hbm.at[wid], score_v)
        pltpu.sync_copy(boxes_hbm.at[wid], boxes_v)
        lane = lax.iota(jnp.int32, 16)
        zf16 = jnp.zeros((16,), jnp.float32)

        def zinit_ob(i, _):
            ob_v[pl.ds(i * 16, 16)] = zf16
            return 0

        lax.fori_loop(0, (KPAD * 4) // 16, zinit_ob, 0)

        def zinit_os(i, _):
            os_v[pl.ds(i * 16, 16)] = zf16
            return 0

        lax.fori_loop(0, KPAD // 16, zinit_os, 0)

        def p0(i, carry):
            bv, bi = carry
            base = i * 16
            p = score_v[pl.ds(base, 16)]
            y1 = boxes_v[0, pl.ds(base, 16)]
            x1 = boxes_v[1, pl.ds(base, 16)]
            y2 = boxes_v[2, pl.ds(base, 16)]
            x2 = boxes_v[3, pl.ds(base, 16)]
            a2 = jnp.maximum(y2 - y1, 0.0) * jnp.maximum(x2 - x1, 0.0)
            a2_v[pl.ds(base, 16)] = a2
            s = jnp.where(p > SCORE_THRESH, p, NEG)
            score_v[pl.ds(base, 16)] = s
            upd = s > bv
            bv = jnp.where(upd, s, bv)
            bi = jnp.where(upd, base + lane, bi)
            return bv, bi

        bv0 = jnp.full((16,), NEG, jnp.float32)
        bi0 = jnp.zeros((16,), jnp.int32)
        bv, bi = lax.fori_loop(0, NCHUNK, p0, (bv0, bi0))
        m0 = jnp.max(bv)
        sel0 = jnp.min(jnp.where(bv >= m0, bi, NPAD))

        def cond(carry):
            k, m, sel = carry
            return (k < K) & (m > 0.0)

        def body(carry):
            k, m, sel = carry
            selv = jnp.full((16,), sel, jnp.int32)
            by1 = plsc.load_gather(boxes_v, [jnp.zeros((16,), jnp.int32), selv])
            bx1 = plsc.load_gather(boxes_v, [jnp.full((16,), 1, jnp.int32), selv])
            by2 = plsc.load_gather(boxes_v, [jnp.full((16,), 2, jnp.int32), selv])
            bx2 = plsc.load_gather(boxes_v, [jnp.full((16,), 3, jnp.int32), selv])
            a1 = jnp.maximum(by2 - by1, 0.0) * jnp.maximum(bx2 - bx1, 0.0)

            boxvec = jnp.where(lane == 0, by1,
                               jnp.where(lane == 1, bx1,
                                         jnp.where(lane == 2, by2, bx2)))
            plsc.store_scatter(ob_v, [k * 4 + lane], boxvec, mask=lane < 4)
            plsc.store_scatter(os_v, [jnp.full((16,), k, jnp.int32)],
                               jnp.full((16,), m, jnp.float32), mask=lane == 0)

            def step(i, carry2):
                nbv, nbi = carry2
                base = i * 16
                s = score_v[pl.ds(base, 16)]
                y1 = boxes_v[0, pl.ds(base, 16)]
                x1 = boxes_v[1, pl.ds(base, 16)]
                y2 = boxes_v[2, pl.ds(base, 16)]
                x2 = boxes_v[3, pl.ds(base, 16)]
                tl_y = jnp.maximum(by1, y1)
                tl_x = jnp.maximum(bx1, x1)
                br_y = jnp.minimum(by2, y2)
                br_x = jnp.minimum(bx2, x2)
                wh_y = jnp.maximum(br_y - tl_y, 0.0)
                wh_x = jnp.maximum(br_x - tl_x, 0.0)
                inter = wh_y * wh_x
                a2 = a2_v[pl.ds(base, 16)]
                iou = inter / (a1 + a2 - inter + 1e-9)
                idxv = base + lane
                sup = (iou > NMS_THRESH) | (idxv == sel)
                ns = jnp.where(sup, NEG, s)
                score_v[pl.ds(base, 16)] = ns
                upd = ns > nbv
                nbv = jnp.where(upd, ns, nbv)
                nbi = jnp.where(upd, idxv, nbi)
                return nbv, nbi

            nbv, nbi = lax.fori_loop(0, NCHUNK, step, (bv0, bi0))
            m2 = jnp.max(nbv)
            sel2 = jnp.min(jnp.where(nbv >= m2, nbi, NPAD))
            return k + 1, m2, sel2

        kfin, _, _ = lax.while_loop(cond, body, (jnp.int32(0), m0, sel0))

        def lfill(i, _):
            base = i * 16
            ol_v[pl.ds(base, 16)] = jnp.where(base + lane < kfin, wid, -1)
            return 0

        lax.fori_loop(0, KPAD // 16, lfill, 0)

        pltpu.sync_copy(ob_v, ob_hbm.at[wid])
        pltpu.sync_copy(os_v, os_hbm.at[wid])
        pltpu.sync_copy(ol_v, ol_hbm.at[wid])


@jax.jit
def kernel(roi_cls_loc, roi_scores, rois):
    pad = NPAD - N
    scores_t = jnp.pad(roi_scores, ((0, pad), (0, 0))).T
    loc_t = jnp.transpose(
        jnp.pad(roi_cls_loc.reshape(N, NCLS, 4), ((0, pad), (0, 0), (0, 0))),
        (1, 2, 0))
    rois_t = jnp.pad(rois, ((0, pad), (0, 0))).T

    probs, boxes = pl.pallas_call(
        _prep_body,
        out_shape=[
            jax.ShapeDtypeStruct((NFG, NPAD), jnp.float32),
            jax.ShapeDtypeStruct((NFG, 4, NPAD), jnp.float32),
        ],
    )(scores_t, loc_t, rois_t)

    nms = pl.kernel(
        _nms_body,
        out_type=[
            jax.ShapeDtypeStruct((NFG, KPAD * 4), jnp.float32),
            jax.ShapeDtypeStruct((NFG, KPAD), jnp.float32),
            jax.ShapeDtypeStruct((NFG, KPAD), jnp.int32),
        ],
        mesh=plsc.VectorSubcoreMesh(core_axis_name="c", subcore_axis_name="s"),
        compiler_params=pltpu.CompilerParams(needs_layout_passes=False),
        scratch_types=[
            pltpu.VMEM((NPAD,), jnp.float32),
            pltpu.VMEM((4, NPAD), jnp.float32),
            pltpu.VMEM((NPAD,), jnp.float32),
            pltpu.VMEM((KPAD * 4,), jnp.float32),
            pltpu.VMEM((KPAD,), jnp.float32),
            pltpu.VMEM((KPAD,), jnp.int32),
        ],
    )
    ob, os_, ol = nms(probs, boxes)

    out_boxes = ob.reshape(NFG, KPAD, 4)[:, :K, :].reshape(-1, 4)
    out_scores = os_[:, :K].reshape(-1)
    out_labels = ol[:, :K].reshape(-1)
    return out_boxes, out_labels, out_scores


# speedup vs baseline: 21.5092x; 3.6430x over previous
import functools

import jax
import jax.numpy as jnp
from jax import lax
from jax.experimental import pallas as pl
from jax.experimental.pallas import tpu as pltpu, tpu_sc as plsc

N = 5000
NPAD = 5120
NCHUNK = NPAD // 16
NCLS = 21
NFG = NCLS - 1
K = 100
KPAD = 128
NMS_THRESH = 0.3
SCORE_THRESH = 0.05
IMG_H = 600.0
IMG_W = 800.0
NEG = -1.0


def _prep_body(scores_ref, loc_ref, rois_ref, probs_ref, boxes_ref):
    scores = scores_ref[...]
    mx = jnp.max(scores, axis=0, keepdims=True)
    e = jnp.exp(scores - mx)
    denom = jnp.sum(e, axis=0, keepdims=True)
    col = lax.broadcasted_iota(jnp.int32, (1, NPAD), 1)
    row_valid = col < N

    src_h = rois_ref[2:3, :] - rois_ref[0:1, :]
    src_w = rois_ref[3:4, :] - rois_ref[1:2, :]
    ctr_y = rois_ref[0:1, :] + 0.5 * src_h
    ctr_x = rois_ref[1:2, :] + 0.5 * src_w

    for c in range(1, NCLS):
        dy = loc_ref[c, 0:1, :] * 0.1
        dx = loc_ref[c, 1:2, :] * 0.1
        dh = loc_ref[c, 2:3, :] * 0.2
        dw = loc_ref[c, 3:4, :] * 0.2
        h = jnp.exp(dh) * src_h
        w = jnp.exp(dw) * src_w
        cy = dy * src_h + ctr_y
        cx = dx * src_w + ctr_x
        y1 = jnp.clip(cy - 0.5 * h, 0.0, IMG_H)
        x1 = jnp.clip(cx - 0.5 * w, 0.0, IMG_W)
        y2 = jnp.clip(cy + 0.5 * h, 0.0, IMG_H)
        x2 = jnp.clip(cx + 0.5 * w, 0.0, IMG_W)
        boxes_ref[c - 1, 0:1, :] = y1
        boxes_ref[c - 1, 1:2, :] = x1
        boxes_ref[c - 1, 2:3, :] = y2
        boxes_ref[c - 1, 3:4, :] = x2
        prob = e[c : c + 1, :] / denom
        probs_ref[c - 1 : c, :] = jnp.where(row_valid, prob, 0.0)


def _nms_body(probs_hbm, boxes_hbm, ob_hbm, os_hbm, ol_hbm,
              score_v, boxes_v, a2_v, ob_v, os_v, ol_v):
    cidx = lax.axis_index("c")
    sidx = lax.axis_index("s")
    wid = sidx * 2 + cidx

    @pl.when(wid < NFG)
    def _work():
        pltpu.sync_copy(probs_hbm.at[wid], score_v.at[pl.ds(0, NPAD)])
        pltpu.sync_copy(boxes_hbm.at[wid], boxes_v)
        lane = lax.iota(jnp.int32, 16)
        zf16 = jnp.zeros((16,), jnp.float32)

        def zinit_ob(i, _):
            ob_v[pl.ds(i * 16, 16)] = zf16
            return 0

        lax.fori_loop(0, (KPAD * 4) // 16, zinit_ob, 0)

        def zinit_os(i, _):
            os_v[pl.ds(i * 16, 16)] = zf16
            return 0

        lax.fori_loop(0, KPAD // 16, zinit_os, 0)

        def p0(i, carry):
            bv, bi, off = carry
            base = i * 16
            p = score_v[pl.ds(base, 16)]
            y1 = boxes_v[0, pl.ds(base, 16)]
            x1 = boxes_v[1, pl.ds(base, 16)]
            y2 = boxes_v[2, pl.ds(base, 16)]
            x2 = boxes_v[3, pl.ds(base, 16)]
            a2 = jnp.maximum(y2 - y1, 0.0) * jnp.maximum(x2 - x1, 0.0)
            valid = p > SCORE_THRESH
            pos = off + plsc.cumsum(valid.astype(jnp.int32)) - 1
            zi = jnp.zeros((16,), jnp.int32)
            plsc.store_scatter(score_v, [pos], p, mask=valid)
            plsc.store_scatter(boxes_v, [zi, pos], y1, mask=valid)
            plsc.store_scatter(boxes_v, [zi + 1, pos], x1, mask=valid)
            plsc.store_scatter(boxes_v, [zi + 2, pos], y2, mask=valid)
            plsc.store_scatter(boxes_v, [zi + 3, pos], x2, mask=valid)
            plsc.store_scatter(a2_v, [pos], a2, mask=valid)
            upd = valid & (p > bv)
            bv = jnp.where(upd, p, bv)
            bi = jnp.where(upd, pos, bi)
            off = off + plsc.all_reduce_population_count(valid)
            return bv, bi, off

        bv0 = jnp.full((16,), NEG, jnp.float32)
        bi0 = jnp.zeros((16,), jnp.int32)
        bv, bi, off = lax.fori_loop(
            0, NCHUNK, p0, (bv0, bi0, jnp.zeros((16,), jnp.int32)))
        nvalid = jnp.max(off)
        score_v[pl.ds(nvalid, 16)] = jnp.full((16,), NEG, jnp.float32)
        nchunks = (nvalid + 15) // 16
        m0 = jnp.max(bv)
        sel0 = jnp.min(jnp.where(bv >= m0, bi, NPAD))

        def cond(carry):
            k, m, sel = carry
            return (k < K) & (m > 0.0)

        def body(carry):
            k, m, sel = carry
            selv = jnp.full((16,), sel, jnp.int32)
            by1 = plsc.load_gather(boxes_v, [jnp.zeros((16,), jnp.int32), selv])
            bx1 = plsc.load_gather(boxes_v, [jnp.full((16,), 1, jnp.int32), selv])
            by2 = plsc.load_gather(boxes_v, [jnp.full((16,), 2, jnp.int32), selv])
            bx2 = plsc.load_gather(boxes_v, [jnp.full((16,), 3, jnp.int32), selv])
            a1 = jnp.maximum(by2 - by1, 0.0) * jnp.maximum(bx2 - bx1, 0.0)

            boxvec = jnp.where(lane == 0, by1,
                               jnp.where(lane == 1, bx1,
                                         jnp.where(lane == 2, by2, bx2)))
            plsc.store_scatter(ob_v, [k * 4 + lane], boxvec, mask=lane < 4)
            plsc.store_scatter(os_v, [jnp.full((16,), k, jnp.int32)],
                               jnp.full((16,), m, jnp.float32), mask=lane == 0)

            def step(i, carry2):
                nbv, nbi = carry2
                base = i * 16
                s = score_v[pl.ds(base, 16)]
                y1 = boxes_v[0, pl.ds(base, 16)]
                x1 = boxes_v[1, pl.ds(base, 16)]
                y2 = boxes_v[2, pl.ds(base, 16)]
                x2 = boxes_v[3, pl.ds(base, 16)]
                tl_y = jnp.maximum(by1, y1)
                tl_x = jnp.maximum(bx1, x1)
                br_y = jnp.minimum(by2, y2)
                br_x = jnp.minimum(bx2, x2)
                wh_y = jnp.maximum(br_y - tl_y, 0.0)
                wh_x = jnp.maximum(br_x - tl_x, 0.0)
                inter = wh_y * wh_x
                a2 = a2_v[pl.ds(base, 16)]
                iou = inter / (a1 + a2 - inter + 1e-9)
                idxv = base + lane
                sup = (iou > NMS_THRESH) | (idxv == sel)
                ns = jnp.where(sup, NEG, s)
                score_v[pl.ds(base, 16)] = ns
                upd = ns > nbv
                nbv = jnp.where(upd, ns, nbv)
                nbi = jnp.where(upd, idxv, nbi)
                return nbv, nbi

            nbv, nbi = lax.fori_loop(0, nchunks, step, (bv0, bi0))
            m2 = jnp.max(nbv)
            sel2 = jnp.min(jnp.where(nbv >= m2, nbi, NPAD))
            return k + 1, m2, sel2

        kfin, _, _ = lax.while_loop(cond, body, (jnp.int32(0), m0, sel0))

        def lfill(i, _):
            base = i * 16
            ol_v[pl.ds(base, 16)] = jnp.where(base + lane < kfin, wid, -1)
            return 0

        lax.fori_loop(0, KPAD // 16, lfill, 0)

        pltpu.sync_copy(ob_v, ob_hbm.at[wid])
        pltpu.sync_copy(os_v, os_hbm.at[wid])
        pltpu.sync_copy(ol_v, ol_hbm.at[wid])


@jax.jit
def kernel(roi_cls_loc, roi_scores, rois):
    pad = NPAD - N
    scores_t = jnp.pad(roi_scores, ((0, pad), (0, 0))).T
    loc_t = jnp.transpose(
        jnp.pad(roi_cls_loc.reshape(N, NCLS, 4), ((0, pad), (0, 0), (0, 0))),
        (1, 2, 0))
    rois_t = jnp.pad(rois, ((0, pad), (0, 0))).T

    probs, boxes = pl.pallas_call(
        _prep_body,
        out_shape=[
            jax.ShapeDtypeStruct((NFG, NPAD), jnp.float32),
            jax.ShapeDtypeStruct((NFG, 4, NPAD), jnp.float32),
        ],
    )(scores_t, loc_t, rois_t)

    nms = pl.kernel(
        _nms_body,
        out_type=[
            jax.ShapeDtypeStruct((NFG, KPAD * 4), jnp.float32),
            jax.ShapeDtypeStruct((NFG, KPAD), jnp.float32),
            jax.ShapeDtypeStruct((NFG, KPAD), jnp.int32),
        ],
        mesh=plsc.VectorSubcoreMesh(core_axis_name="c", subcore_axis_name="s"),
        compiler_params=pltpu.CompilerParams(needs_layout_passes=False),
        scratch_types=[
            pltpu.VMEM((NPAD + 16,), jnp.float32),
            pltpu.VMEM((4, NPAD), jnp.float32),
            pltpu.VMEM((NPAD,), jnp.float32),
            pltpu.VMEM((KPAD * 4,), jnp.float32),
            pltpu.VMEM((KPAD,), jnp.float32),
            pltpu.VMEM((KPAD,), jnp.int32),
        ],
    )
    ob, os_, ol = nms(probs, boxes)

    out_boxes = ob.reshape(NFG, KPAD, 4)[:, :K, :].reshape(-1, 4)
    out_scores = os_[:, :K].reshape(-1)
    out_labels = ol[:, :K].reshape(-1)
    return out_boxes, out_labels, out_scores


# speedup vs baseline: 49.4537x; 2.2992x over previous
import functools

import jax
import jax.numpy as jnp
from jax import lax
from jax.experimental import pallas as pl
from jax.experimental.pallas import tpu as pltpu, tpu_sc as plsc

N = 5000
NPAD = 5120
NCHUNK = NPAD // 16
NCLS = 21
NFG = NCLS - 1
K = 100
KPAD = 128
NMS_THRESH = 0.3
SCORE_THRESH = 0.05
IMG_H = 600.0
IMG_W = 800.0
NEG = -1.0


def _prep_body(scores_ref, loc_ref, rois_ref, probs_ref, boxes_ref):
    scores = scores_ref[...]
    mx = jnp.max(scores, axis=0, keepdims=True)
    e = jnp.exp(scores - mx)
    denom = jnp.sum(e, axis=0, keepdims=True)
    col = lax.broadcasted_iota(jnp.int32, (1, NPAD), 1)
    row_valid = col < N

    src_h = rois_ref[2:3, :] - rois_ref[0:1, :]
    src_w = rois_ref[3:4, :] - rois_ref[1:2, :]
    ctr_y = rois_ref[0:1, :] + 0.5 * src_h
    ctr_x = rois_ref[1:2, :] + 0.5 * src_w

    for c in range(1, NCLS):
        dy = loc_ref[c, 0:1, :] * 0.1
        dx = loc_ref[c, 1:2, :] * 0.1
        dh = loc_ref[c, 2:3, :] * 0.2
        dw = loc_ref[c, 3:4, :] * 0.2
        h = jnp.exp(dh) * src_h
        w = jnp.exp(dw) * src_w
        cy = dy * src_h + ctr_y
        cx = dx * src_w + ctr_x
        y1 = jnp.clip(cy - 0.5 * h, 0.0, IMG_H)
        x1 = jnp.clip(cx - 0.5 * w, 0.0, IMG_W)
        y2 = jnp.clip(cy + 0.5 * h, 0.0, IMG_H)
        x2 = jnp.clip(cx + 0.5 * w, 0.0, IMG_W)
        boxes_ref[c - 1, 0:1, :] = y1
        boxes_ref[c - 1, 1:2, :] = x1
        boxes_ref[c - 1, 2:3, :] = y2
        boxes_ref[c - 1, 3:4, :] = x2
        prob = e[c : c + 1, :] / denom
        probs_ref[c - 1 : c, :] = jnp.where(row_valid, prob, 0.0)


def _nms_body(probs_hbm, boxes_hbm, ob_hbm, os_hbm, ol_hbm,
              score_v, boxes_v, a2_v, ob_v, os_v, ol_v):
    cidx = lax.axis_index("c")
    sidx = lax.axis_index("s")
    wid = sidx * 2 + cidx

    @pl.when(wid < NFG)
    def _work():
        pltpu.sync_copy(probs_hbm.at[wid], score_v.at[pl.ds(0, NPAD)])
        pltpu.sync_copy(boxes_hbm.at[wid], boxes_v)
        lane = lax.iota(jnp.int32, 16)
        zf16 = jnp.zeros((16,), jnp.float32)

        def zinit_ob(i, _):
            ob_v[pl.ds(i * 16, 16)] = zf16
            return 0

        lax.fori_loop(0, (KPAD * 4) // 16, zinit_ob, 0)

        def zinit_os(i, _):
            os_v[pl.ds(i * 16, 16)] = zf16
            return 0

        lax.fori_loop(0, KPAD // 16, zinit_os, 0)

        def p0(i, carry):
            bv, bi, off = carry
            base = i * 16
            p = score_v[pl.ds(base, 16)]
            y1 = boxes_v[0, pl.ds(base, 16)]
            x1 = boxes_v[1, pl.ds(base, 16)]
            y2 = boxes_v[2, pl.ds(base, 16)]
            x2 = boxes_v[3, pl.ds(base, 16)]
            a2 = jnp.maximum(y2 - y1, 0.0) * jnp.maximum(x2 - x1, 0.0)
            valid = p > SCORE_THRESH
            pos = off + plsc.cumsum(valid.astype(jnp.int32)) - 1
            zi = jnp.zeros((16,), jnp.int32)
            plsc.store_scatter(score_v, [pos], p, mask=valid)
            plsc.store_scatter(boxes_v, [zi, pos], y1, mask=valid)
            plsc.store_scatter(boxes_v, [zi + 1, pos], x1, mask=valid)
            plsc.store_scatter(boxes_v, [zi + 2, pos], y2, mask=valid)
            plsc.store_scatter(boxes_v, [zi + 3, pos], x2, mask=valid)
            plsc.store_scatter(a2_v, [pos], a2, mask=valid)
            upd = valid & (p > bv)
            bv = jnp.where(upd, p, bv)
            bi = jnp.where(upd, pos, bi)
            off = off + plsc.all_reduce_population_count(valid)
            return bv, bi, off

        bv0 = jnp.full((16,), NEG, jnp.float32)
        bi0 = jnp.zeros((16,), jnp.int32)
        bv, bi, off = lax.fori_loop(
            0, NCHUNK, p0, (bv0, bi0, jnp.zeros((16,), jnp.int32)))
        nvalid = jnp.max(off)
        score_v[pl.ds(nvalid, 16)] = jnp.full((16,), NEG, jnp.float32)
        nchunks = (nvalid + 15) // 16
        m0 = jnp.max(bv)
        sel0 = jnp.min(jnp.where(bv >= m0, bi, NPAD))

        def cond(carry):
            k, m, sel = carry
            return (k < K) & (m > 0.0)

        def body(carry):
            k, m, sel = carry
            selv = jnp.full((16,), sel, jnp.int32)
            by1 = plsc.load_gather(boxes_v, [jnp.zeros((16,), jnp.int32), selv])
            bx1 = plsc.load_gather(boxes_v, [jnp.full((16,), 1, jnp.int32), selv])
            by2 = plsc.load_gather(boxes_v, [jnp.full((16,), 2, jnp.int32), selv])
            bx2 = plsc.load_gather(boxes_v, [jnp.full((16,), 3, jnp.int32), selv])
            a1 = jnp.maximum(by2 - by1, 0.0) * jnp.maximum(bx2 - bx1, 0.0)

            boxvec = jnp.where(lane == 0, by1,
                               jnp.where(lane == 1, bx1,
                                         jnp.where(lane == 2, by2, bx2)))
            plsc.store_scatter(ob_v, [k * 4 + lane], boxvec, mask=lane < 4)
            plsc.store_scatter(os_v, [jnp.full((16,), k, jnp.int32)],
                               jnp.full((16,), m, jnp.float32), mask=lane == 0)
            plsc.store_scatter(score_v, [selv],
                               jnp.full((16,), NEG, jnp.float32),
                               mask=lane == 0)

            def step(i, carry2):
                nbv, nbi = carry2
                base = i * 16
                s = score_v[pl.ds(base, 16)]
                y1 = boxes_v[0, pl.ds(base, 16)]
                x1 = boxes_v[1, pl.ds(base, 16)]
                y2 = boxes_v[2, pl.ds(base, 16)]
                x2 = boxes_v[3, pl.ds(base, 16)]
                tl_y = jnp.maximum(by1, y1)
                tl_x = jnp.maximum(bx1, x1)
                br_y = jnp.minimum(by2, y2)
                br_x = jnp.minimum(bx2, x2)
                wh_y = jnp.maximum(br_y - tl_y, 0.0)
                wh_x = jnp.maximum(br_x - tl_x, 0.0)
                inter = wh_y * wh_x
                a2 = a2_v[pl.ds(base, 16)]
                iou = inter / (a1 + a2 - inter + 1e-9)
                idxv = base + lane
                sup = iou > NMS_THRESH
                ns = jnp.where(sup, NEG, s)
                score_v[pl.ds(base, 16)] = ns
                upd = ns > nbv
                nbv = jnp.where(upd, ns, nbv)
                nbi = jnp.where(upd, idxv, nbi)
                return nbv, nbi

            def step_p(base, carry2):
                return step(base // 16, carry2)

            nbv, nbi = plsc.parallel_loop(
                0, nchunks * 16, 16, unroll=4, carry=(bv0, bi0))(step_p)
            m2 = jnp.max(nbv)
            sel2 = jnp.min(jnp.where(nbv >= m2, nbi, NPAD))
            return k + 1, m2, sel2

        kfin, _, _ = lax.while_loop(cond, body, (jnp.int32(0), m0, sel0))

        def lfill(i, _):
            base = i * 16
            ol_v[pl.ds(base, 16)] = jnp.where(base + lane < kfin, wid, -1)
            return 0

        lax.fori_loop(0, KPAD // 16, lfill, 0)

        pltpu.sync_copy(ob_v, ob_hbm.at[wid])
        pltpu.sync_copy(os_v, os_hbm.at[wid])
        pltpu.sync_copy(ol_v, ol_hbm.at[wid])


@jax.jit
def kernel(roi_cls_loc, roi_scores, rois):
    pad = NPAD - N
    scores_t = jnp.pad(roi_scores, ((0, pad), (0, 0))).T
    loc_t = jnp.transpose(
        jnp.pad(roi_cls_loc.reshape(N, NCLS, 4), ((0, pad), (0, 0), (0, 0))),
        (1, 2, 0))
    rois_t = jnp.pad(rois, ((0, pad), (0, 0))).T

    probs, boxes = pl.pallas_call(
        _prep_body,
        out_shape=[
            jax.ShapeDtypeStruct((NFG, NPAD), jnp.float32),
            jax.ShapeDtypeStruct((NFG, 4, NPAD), jnp.float32),
        ],
    )(scores_t, loc_t, rois_t)

    nms = pl.kernel(
        _nms_body,
        out_type=[
            jax.ShapeDtypeStruct((NFG, KPAD * 4), jnp.float32),
            jax.ShapeDtypeStruct((NFG, KPAD), jnp.float32),
            jax.ShapeDtypeStruct((NFG, KPAD), jnp.int32),
        ],
        mesh=plsc.VectorSubcoreMesh(core_axis_name="c", subcore_axis_name="s"),
        compiler_params=pltpu.CompilerParams(needs_layout_passes=False),
        scratch_types=[
            pltpu.VMEM((NPAD + 16,), jnp.float32),
            pltpu.VMEM((4, NPAD), jnp.float32),
            pltpu.VMEM((NPAD,), jnp.float32),
            pltpu.VMEM((KPAD * 4,), jnp.float32),
            pltpu.VMEM((KPAD,), jnp.float32),
            pltpu.VMEM((KPAD,), jnp.int32),
        ],
    )
    ob, os_, ol = nms(probs, boxes)

    out_boxes = ob.reshape(NFG, KPAD, 4)[:, :K, :].reshape(-1, 4)
    out_scores = os_[:, :K].reshape(-1)
    out_labels = ol[:, :K].reshape(-1)
    return out_boxes, out_labels, out_scores


# speedup vs baseline: 58.3056x; 1.1790x over previous
import functools

import jax
import jax.numpy as jnp
from jax import lax
from jax.experimental import pallas as pl
from jax.experimental.pallas import tpu as pltpu, tpu_sc as plsc

N = 5000
NPAD = 5120
NCHUNK = NPAD // 16
NCLS = 21
NFG = NCLS - 1
K = 100
KPAD = 128
NMS_THRESH = 0.3
SCORE_THRESH = 0.05
IMG_H = 600.0
IMG_W = 800.0
NEG = -1.0


def _prep_body(scores_ref, loc_ref, rois_ref, probs_ref, boxes_ref):
    scores = scores_ref[...]
    mx = jnp.max(scores, axis=0, keepdims=True)
    e = jnp.exp(scores - mx)
    denom = jnp.sum(e, axis=0, keepdims=True)
    col = lax.broadcasted_iota(jnp.int32, (1, NPAD), 1)
    row_valid = col < N

    src_h = rois_ref[2:3, :] - rois_ref[0:1, :]
    src_w = rois_ref[3:4, :] - rois_ref[1:2, :]
    ctr_y = rois_ref[0:1, :] + 0.5 * src_h
    ctr_x = rois_ref[1:2, :] + 0.5 * src_w

    for c in range(1, NCLS):
        dy = loc_ref[c, 0:1, :] * 0.1
        dx = loc_ref[c, 1:2, :] * 0.1
        dh = loc_ref[c, 2:3, :] * 0.2
        dw = loc_ref[c, 3:4, :] * 0.2
        h = jnp.exp(dh) * src_h
        w = jnp.exp(dw) * src_w
        cy = dy * src_h + ctr_y
        cx = dx * src_w + ctr_x
        y1 = jnp.clip(cy - 0.5 * h, 0.0, IMG_H)
        x1 = jnp.clip(cx - 0.5 * w, 0.0, IMG_W)
        y2 = jnp.clip(cy + 0.5 * h, 0.0, IMG_H)
        x2 = jnp.clip(cx + 0.5 * w, 0.0, IMG_W)
        boxes_ref[c - 1, 0:1, :] = y1
        boxes_ref[c - 1, 1:2, :] = x1
        boxes_ref[c - 1, 2:3, :] = y2
        boxes_ref[c - 1, 3:4, :] = x2
        prob = e[c : c + 1, :] / denom
        probs_ref[c - 1 : c, :] = jnp.where(row_valid, prob, 0.0)


def _nms_body(probs_hbm, boxes_hbm, ob_hbm, os_hbm, ol_hbm,
              score_v, boxes_v, score_c, boxes_c, a2_v, cm_v,
              sel_v, ob_v, os_v, ol_v):
    cidx = lax.axis_index("c")
    sidx = lax.axis_index("s")
    wid = sidx * 2 + cidx

    @pl.when(wid < NFG)
    def _work():
        pltpu.sync_copy(probs_hbm.at[wid], score_v)
        pltpu.sync_copy(boxes_hbm.at[wid], boxes_v)
        lane = lax.iota(jnp.int32, 16)
        zf16 = jnp.zeros((16,), jnp.float32)
        negv = jnp.full((16,), NEG, jnp.float32)

        def zinit_ob(i, _):
            ob_v[pl.ds(i * 16, 16)] = zf16
            return 0

        lax.fori_loop(0, (KPAD * 4) // 16, zinit_ob, 0)

        def zinit_os(i, _):
            os_v[pl.ds(i * 16, 16)] = zf16
            sel_v[0, pl.ds(i * 16, 16)] = zf16
            sel_v[1, pl.ds(i * 16, 16)] = zf16
            sel_v[2, pl.ds(i * 16, 16)] = zf16
            sel_v[3, pl.ds(i * 16, 16)] = zf16
            sel_v[4, pl.ds(i * 16, 16)] = zf16
            return 0

        lax.fori_loop(0, KPAD // 16, zinit_os, 0)

        def p0(i, carry):
            bv, bi, off = carry
            base = i * 16
            p = score_v[pl.ds(base, 16)]
            y1 = boxes_v[0, pl.ds(base, 16)]
            x1 = boxes_v[1, pl.ds(base, 16)]
            y2 = boxes_v[2, pl.ds(base, 16)]
            x2 = boxes_v[3, pl.ds(base, 16)]
            a2 = jnp.maximum(y2 - y1, 0.0) * jnp.maximum(x2 - x1, 0.0)
            valid = p > SCORE_THRESH
            pos = off + plsc.cumsum(valid.astype(jnp.int32)) - 1
            zi = jnp.zeros((16,), jnp.int32)
            plsc.store_scatter(score_c, [pos], p, mask=valid)
            plsc.store_scatter(boxes_c, [zi, pos], y1, mask=valid)
            plsc.store_scatter(boxes_c, [zi + 1, pos], x1, mask=valid)
            plsc.store_scatter(boxes_c, [zi + 2, pos], y2, mask=valid)
            plsc.store_scatter(boxes_c, [zi + 3, pos], x2, mask=valid)
            plsc.store_scatter(a2_v, [pos], a2, mask=valid)
            upd = valid & (p > bv)
            bv = jnp.where(upd, p, bv)
            bi = jnp.where(upd, pos, bi)
            off = off + plsc.all_reduce_population_count(valid)
            return bv, bi, off

        bv0 = jnp.full((16,), NEG, jnp.float32)
        bi0 = jnp.zeros((16,), jnp.int32)
        bv, bi, off = lax.fori_loop(
            0, NCHUNK, p0, (bv0, bi0, jnp.zeros((16,), jnp.int32)))
        nvalid = jnp.max(off)
        nchunks = (nvalid + 15) // 16
        base_last = (nvalid // 16) * 16
        s_last = score_c[pl.ds(base_last, 16)]
        score_c[pl.ds(base_last, 16)] = jnp.where(
            base_last + lane < nvalid, s_last, negv)

        def cmneg(i, _):
            cm_v[pl.ds(i * 16, 16)] = negv
            return 0

        lax.fori_loop(0, (NCHUNK + 16) // 16, cmneg, 0)

        def cmb(j, _):
            s = score_c[pl.ds(j * 16, 16)]
            mx = jnp.where(j < nchunks, jnp.max(s), NEG)
            plsc.store_scatter(cm_v, [jnp.full((16,), j, jnp.int32)],
                               jnp.full((16,), mx, jnp.float32),
                               mask=lane == 0)
            return 0

        lax.fori_loop(0, NCHUNK, cmb, 0)
        ncm = (NCHUNK + 16) // 16

        def cm_argmax(u, c):
            bv, bi = c
            cmv = cm_v[pl.ds(u * 16, 16)]
            upd = cmv > bv
            bv = jnp.where(upd, cmv, bv)
            bi = jnp.where(upd, u * 16 + lane, bi)
            return bv, bi

        m0 = jnp.max(bv)
        j0 = jnp.min(jnp.where(bv >= m0, bi, NPAD)) // 16

        def cond(carry):
            ns, m, jstar = carry
            return (ns < K) & (m > 0.0)

        def body(carry):
            ns, m, jstar = carry
            base = jstar * 16
            s = score_c[pl.ds(base, 16)]
            lsel = jnp.min(jnp.where(s >= m, lane, 16))
            idx = base + lsel
            s2 = jnp.where(lane == lsel, NEG, s)
            score_c[pl.ds(base, 16)] = s2
            plsc.store_scatter(cm_v, [jnp.full((16,), jstar, jnp.int32)],
                               jnp.full((16,), jnp.max(s2), jnp.float32),
                               mask=lane == 0)

            idxv = jnp.full((16,), idx, jnp.int32)
            by1 = plsc.load_gather(boxes_c, [jnp.zeros((16,), jnp.int32), idxv])
            bx1 = plsc.load_gather(boxes_c, [jnp.full((16,), 1, jnp.int32), idxv])
            by2 = plsc.load_gather(boxes_c, [jnp.full((16,), 2, jnp.int32), idxv])
            bx2 = plsc.load_gather(boxes_c, [jnp.full((16,), 3, jnp.int32), idxv])
            ca = plsc.load_gather(a2_v, [idxv])

            def tchk(t, mi):
                tb = t * 16
                sy1 = sel_v[0, pl.ds(tb, 16)]
                sx1 = sel_v[1, pl.ds(tb, 16)]
                sy2 = sel_v[2, pl.ds(tb, 16)]
                sx2 = sel_v[3, pl.ds(tb, 16)]
                sa = sel_v[4, pl.ds(tb, 16)]
                tl_y = jnp.maximum(sy1, by1)
                tl_x = jnp.maximum(sx1, bx1)
                br_y = jnp.minimum(sy2, by2)
                br_x = jnp.minimum(sx2, bx2)
                wh_y = jnp.maximum(br_y - tl_y, 0.0)
                wh_x = jnp.maximum(br_x - tl_x, 0.0)
                inter = wh_y * wh_x
                iou = inter / (sa + ca - inter + 1e-9)
                return jnp.maximum(mi, iou)

            maxiou_v = lax.fori_loop(0, (ns + 15) // 16, tchk, zf16)
            keep = jnp.max(maxiou_v) <= NMS_THRESH

            @pl.when(keep)
            def _acc():
                nsv = jnp.full((16,), ns, jnp.int32)
                lane0 = lane == 0
                zi0 = jnp.zeros((16,), jnp.int32)
                plsc.store_scatter(sel_v, [zi0, nsv], by1, mask=lane0)
                plsc.store_scatter(sel_v, [zi0 + 1, nsv], bx1, mask=lane0)
                plsc.store_scatter(sel_v, [zi0 + 2, nsv], by2, mask=lane0)
                plsc.store_scatter(sel_v, [zi0 + 3, nsv], bx2, mask=lane0)
                plsc.store_scatter(sel_v, [zi0 + 4, nsv], ca, mask=lane0)
                boxvec = jnp.where(lane == 0, by1,
                                   jnp.where(lane == 1, bx1,
                                             jnp.where(lane == 2, by2, bx2)))
                plsc.store_scatter(ob_v, [ns * 4 + lane], boxvec,
                                   mask=lane < 4)
                plsc.store_scatter(os_v, [nsv],
                                   jnp.full((16,), m, jnp.float32),
                                   mask=lane0)

            ns2 = jnp.where(keep, ns + 1, ns)
            bv, bi = lax.fori_loop(0, ncm, cm_argmax, (bv0, bi0))
            m2 = jnp.max(bv)
            j2 = jnp.min(jnp.where(bv >= m2, bi, NCHUNK + 16))
            return ns2, m2, j2

        kfin, _, _ = lax.while_loop(cond, body, (jnp.int32(0), m0, j0))

        def lfill(i, _):
            base = i * 16
            ol_v[pl.ds(base, 16)] = jnp.where(base + lane < kfin, wid, -1)
            return 0

        lax.fori_loop(0, KPAD // 16, lfill, 0)

        pltpu.sync_copy(ob_v, ob_hbm.at[wid])
        pltpu.sync_copy(os_v, os_hbm.at[wid])
        pltpu.sync_copy(ol_v, ol_hbm.at[wid])


@jax.jit
def kernel(roi_cls_loc, roi_scores, rois):
    pad = NPAD - N
    scores_t = jnp.pad(roi_scores, ((0, pad), (0, 0))).T
    loc_t = jnp.transpose(
        jnp.pad(roi_cls_loc.reshape(N, NCLS, 4), ((0, pad), (0, 0), (0, 0))),
        (1, 2, 0))
    rois_t = jnp.pad(rois, ((0, pad), (0, 0))).T

    probs, boxes = pl.pallas_call(
        _prep_body,
        out_shape=[
            jax.ShapeDtypeStruct((NFG, NPAD), jnp.float32),
            jax.ShapeDtypeStruct((NFG, 4, NPAD), jnp.float32),
        ],
    )(scores_t, loc_t, rois_t)

    nms = pl.kernel(
        _nms_body,
        out_type=[
            jax.ShapeDtypeStruct((NFG, KPAD * 4), jnp.float32),
            jax.ShapeDtypeStruct((NFG, KPAD), jnp.float32),
            jax.ShapeDtypeStruct((NFG, KPAD), jnp.int32),
        ],
        mesh=plsc.VectorSubcoreMesh(core_axis_name="c", subcore_axis_name="s"),
        compiler_params=pltpu.CompilerParams(needs_layout_passes=False),
        scratch_types=[
            pltpu.VMEM((NPAD,), jnp.float32),
            pltpu.VMEM((4, NPAD), jnp.float32),
            pltpu.VMEM((NPAD + 16,), jnp.float32),
            pltpu.VMEM((4, NPAD), jnp.float32),
            pltpu.VMEM((NPAD,), jnp.float32),
            pltpu.VMEM((NCHUNK + 16,), jnp.float32),
            pltpu.VMEM((5, KPAD), jnp.float32),
            pltpu.VMEM((KPAD * 4,), jnp.float32),
            pltpu.VMEM((KPAD,), jnp.float32),
            pltpu.VMEM((KPAD,), jnp.int32),
        ],
    )
    ob, os_, ol = nms(probs, boxes)

    out_boxes = ob.reshape(NFG, KPAD, 4)[:, :K, :].reshape(-1, 4)
    out_scores = os_[:, :K].reshape(-1)
    out_labels = ol[:, :K].reshape(-1)
    return out_boxes, out_labels, out_scores


# speedup vs baseline: 62.3236x; 1.0689x over previous
import functools

import jax
import jax.numpy as jnp
from jax import lax
from jax.experimental import pallas as pl
from jax.experimental.pallas import tpu as pltpu, tpu_sc as plsc

N = 5000
NPAD = 5120
NCHUNK = NPAD // 16
NCLS = 21
NFG = NCLS - 1
K = 100
KPAD = 128
NMS_THRESH = 0.3
SCORE_THRESH = 0.05
IMG_H = 600.0
IMG_W = 800.0
NEG = -1.0


def _prep_body(scores_ref, loc_ref, rois_ref, probs_ref, boxes_ref):
    scores = scores_ref[...]
    mx = jnp.max(scores, axis=0, keepdims=True)
    e = jnp.exp(scores - mx)
    denom = jnp.sum(e, axis=0, keepdims=True)
    col = lax.broadcasted_iota(jnp.int32, (1, NPAD), 1)
    row_valid = col < N
    probs_ref[...] = jnp.where(row_valid, e / denom, 0.0)

    src_h = rois_ref[2:3, :] - rois_ref[0:1, :]
    src_w = rois_ref[3:4, :] - rois_ref[1:2, :]
    ctr_y = rois_ref[0:1, :] + 0.5 * src_h
    ctr_x = rois_ref[1:2, :] + 0.5 * src_w

    dy = loc_ref[0] * 0.1
    dx = loc_ref[1] * 0.1
    dh = loc_ref[2] * 0.2
    dw = loc_ref[3] * 0.2
    h = jnp.exp(dh) * src_h
    w = jnp.exp(dw) * src_w
    cy = dy * src_h + ctr_y
    cx = dx * src_w + ctr_x
    boxes_ref[0] = jnp.clip(cy - 0.5 * h, 0.0, IMG_H)
    boxes_ref[1] = jnp.clip(cx - 0.5 * w, 0.0, IMG_W)
    boxes_ref[2] = jnp.clip(cy + 0.5 * h, 0.0, IMG_H)
    boxes_ref[3] = jnp.clip(cx + 0.5 * w, 0.0, IMG_W)


def _nms_body(probs_hbm, boxes_hbm, ob_hbm, os_hbm, ol_hbm,
              score_v, boxes_v, cm_v, sel_v, ob_v, os_v, ol_v):
    cidx = lax.axis_index("c")
    sidx = lax.axis_index("s")
    wid = sidx * 2 + cidx

    @pl.when(wid < NFG)
    def _work():
        cls = wid + 1
        pltpu.sync_copy(probs_hbm.at[cls], score_v)
        pltpu.sync_copy(boxes_hbm.at[0, cls], boxes_v.at[0])
        pltpu.sync_copy(boxes_hbm.at[1, cls], boxes_v.at[1])
        pltpu.sync_copy(boxes_hbm.at[2, cls], boxes_v.at[2])
        pltpu.sync_copy(boxes_hbm.at[3, cls], boxes_v.at[3])
        lane = lax.iota(jnp.int32, 16)
        zf16 = jnp.zeros((16,), jnp.float32)
        negv = jnp.full((16,), NEG, jnp.float32)

        def zinit_ob(i, _):
            ob_v[pl.ds(i * 16, 16)] = zf16
            return 0

        lax.fori_loop(0, (KPAD * 4) // 16, zinit_ob, 0)

        def zinit_os(i, _):
            os_v[pl.ds(i * 16, 16)] = zf16
            sel_v[0, pl.ds(i * 16, 16)] = zf16
            sel_v[1, pl.ds(i * 16, 16)] = zf16
            sel_v[2, pl.ds(i * 16, 16)] = zf16
            sel_v[3, pl.ds(i * 16, 16)] = zf16
            sel_v[4, pl.ds(i * 16, 16)] = zf16
            return 0

        lax.fori_loop(0, KPAD // 16, zinit_os, 0)

        def cmneg(i, _):
            cm_v[pl.ds(i * 16, 16)] = negv
            return 0

        lax.fori_loop(0, (NCHUNK + 16) // 16, cmneg, 0)

        def cmb(j, _):
            p = score_v[pl.ds(j * 16, 16)]
            s = jnp.where(p > SCORE_THRESH, p, NEG)
            score_v[pl.ds(j * 16, 16)] = s
            plsc.store_scatter(cm_v, [jnp.full((16,), j, jnp.int32)],
                               jnp.full((16,), jnp.max(s), jnp.float32),
                               mask=lane == 0)
            return 0

        lax.fori_loop(0, NCHUNK, cmb, 0)
        ncm = (NCHUNK + 16) // 16

        bv0 = jnp.full((16,), NEG, jnp.float32)
        bi0 = jnp.zeros((16,), jnp.int32)

        def cm_argmax(u, c):
            bv, bi = c
            cmv = cm_v[pl.ds(u * 16, 16)]
            upd = cmv > bv
            bv = jnp.where(upd, cmv, bv)
            bi = jnp.where(upd, u * 16 + lane, bi)
            return bv, bi

        bv, bi = lax.fori_loop(0, ncm, cm_argmax, (bv0, bi0))
        m0 = jnp.max(bv)
        j0 = jnp.min(jnp.where(bv >= m0, bi, NCHUNK + 16))

        def cond(carry):
            ns, m, jstar = carry
            return (ns < K) & (m > 0.0)

        def body(carry):
            ns, m, jstar = carry
            base = jstar * 16
            s = score_v[pl.ds(base, 16)]
            lsel = jnp.min(jnp.where(s >= m, lane, 16))
            idx = base + lsel
            s2 = jnp.where(lane == lsel, NEG, s)
            score_v[pl.ds(base, 16)] = s2
            plsc.store_scatter(cm_v, [jnp.full((16,), jstar, jnp.int32)],
                               jnp.full((16,), jnp.max(s2), jnp.float32),
                               mask=lane == 0)

            idxv = jnp.full((16,), idx, jnp.int32)
            by1 = plsc.load_gather(boxes_v, [jnp.zeros((16,), jnp.int32), idxv])
            bx1 = plsc.load_gather(boxes_v, [jnp.full((16,), 1, jnp.int32), idxv])
            by2 = plsc.load_gather(boxes_v, [jnp.full((16,), 2, jnp.int32), idxv])
            bx2 = plsc.load_gather(boxes_v, [jnp.full((16,), 3, jnp.int32), idxv])
            ca = jnp.maximum(by2 - by1, 0.0) * jnp.maximum(bx2 - bx1, 0.0)

            def tchk(t, mi):
                tb = t * 16
                sy1 = sel_v[0, pl.ds(tb, 16)]
                sx1 = sel_v[1, pl.ds(tb, 16)]
                sy2 = sel_v[2, pl.ds(tb, 16)]
                sx2 = sel_v[3, pl.ds(tb, 16)]
                sa = sel_v[4, pl.ds(tb, 16)]
                tl_y = jnp.maximum(sy1, by1)
                tl_x = jnp.maximum(sx1, bx1)
                br_y = jnp.minimum(sy2, by2)
                br_x = jnp.minimum(sx2, bx2)
                wh_y = jnp.maximum(br_y - tl_y, 0.0)
                wh_x = jnp.maximum(br_x - tl_x, 0.0)
                inter = wh_y * wh_x
                iou = inter / (sa + ca - inter + 1e-9)
                return jnp.maximum(mi, iou)

            maxiou_v = lax.fori_loop(0, (ns + 15) // 16, tchk, zf16)
            keep = jnp.max(maxiou_v) <= NMS_THRESH

            @pl.when(keep)
            def _acc():
                nsv = jnp.full((16,), ns, jnp.int32)
                lane0 = lane == 0
                zi0 = jnp.zeros((16,), jnp.int32)
                plsc.store_scatter(sel_v, [zi0, nsv], by1, mask=lane0)
                plsc.store_scatter(sel_v, [zi0 + 1, nsv], bx1, mask=lane0)
                plsc.store_scatter(sel_v, [zi0 + 2, nsv], by2, mask=lane0)
                plsc.store_scatter(sel_v, [zi0 + 3, nsv], bx2, mask=lane0)
                plsc.store_scatter(sel_v, [zi0 + 4, nsv], ca, mask=lane0)
                boxvec = jnp.where(lane == 0, by1,
                                   jnp.where(lane == 1, bx1,
                                             jnp.where(lane == 2, by2, bx2)))
                plsc.store_scatter(ob_v, [ns * 4 + lane], boxvec,
                                   mask=lane < 4)
                plsc.store_scatter(os_v, [nsv],
                                   jnp.full((16,), m, jnp.float32),
                                   mask=lane0)

            ns2 = jnp.where(keep, ns + 1, ns)
            bv, bi = lax.fori_loop(0, ncm, cm_argmax, (bv0, bi0))
            m2 = jnp.max(bv)
            j2 = jnp.min(jnp.where(bv >= m2, bi, NCHUNK + 16))
            return ns2, m2, j2

        kfin, _, _ = lax.while_loop(cond, body, (jnp.int32(0), m0, j0))

        def lfill(i, _):
            base = i * 16
            ol_v[pl.ds(base, 16)] = jnp.where(base + lane < kfin, wid, -1)
            return 0

        lax.fori_loop(0, KPAD // 16, lfill, 0)

        pltpu.sync_copy(ob_v, ob_hbm.at[wid])
        pltpu.sync_copy(os_v, os_hbm.at[wid])
        pltpu.sync_copy(ol_v, ol_hbm.at[wid])


@jax.jit
def kernel(roi_cls_loc, roi_scores, rois):
    pad = NPAD - N
    scores_t = jnp.pad(roi_scores, ((0, pad), (0, 0))).T
    loc_t = jnp.transpose(
        jnp.pad(roi_cls_loc.reshape(N, NCLS, 4), ((0, pad), (0, 0), (0, 0))),
        (2, 1, 0))
    rois_t = jnp.pad(rois, ((0, pad), (0, 0))).T

    probs, boxes = pl.pallas_call(
        _prep_body,
        out_shape=[
            jax.ShapeDtypeStruct((NCLS, NPAD), jnp.float32),
            jax.ShapeDtypeStruct((4, NCLS, NPAD), jnp.float32),
        ],
    )(scores_t, loc_t, rois_t)

    nms = pl.kernel(
        _nms_body,
        out_type=[
            jax.ShapeDtypeStruct((NFG, KPAD * 4), jnp.float32),
            jax.ShapeDtypeStruct((NFG, KPAD), jnp.float32),
            jax.ShapeDtypeStruct((NFG, KPAD), jnp.int32),
        ],
        mesh=plsc.VectorSubcoreMesh(core_axis_name="c", subcore_axis_name="s"),
        compiler_params=pltpu.CompilerParams(needs_layout_passes=False),
        scratch_types=[
            pltpu.VMEM((NPAD,), jnp.float32),
            pltpu.VMEM((4, NPAD), jnp.float32),
            pltpu.VMEM((NCHUNK + 16,), jnp.float32),
            pltpu.VMEM((5, KPAD), jnp.float32),
            pltpu.VMEM((KPAD * 4,), jnp.float32),
            pltpu.VMEM((KPAD,), jnp.float32),
            pltpu.VMEM((KPAD,), jnp.int32),
        ],
    )
    ob, os_, ol = nms(probs, boxes)

    out_boxes = ob.reshape(NFG, KPAD, 4)[:, :K, :].reshape(-1, 4)
    out_scores = os_[:, :K].reshape(-1)
    out_labels = ol[:, :K].reshape(-1)
    return out_boxes, out_labels, out_scores


# speedup vs baseline: 70.7580x; 1.1353x over previous
import functools

import jax
import jax.numpy as jnp
from jax import lax
from jax.experimental import pallas as pl
from jax.experimental.pallas import tpu as pltpu, tpu_sc as plsc

N = 5000
NPAD = 5120
NCHUNK = NPAD // 16
NCLS = 21
NFG = NCLS - 1
K = 100
KPAD = 128
NMS_THRESH = 0.3
SCORE_THRESH = 0.05
IMG_H = 600.0
IMG_W = 800.0
NEG = -1.0


def _prep_body(scores_ref, loc_ref, rois_ref, probs_ref, boxes_ref):
    scores = scores_ref[...]
    mx = jnp.max(scores, axis=0, keepdims=True)
    e = jnp.exp(scores - mx)
    denom = jnp.sum(e, axis=0, keepdims=True)
    col = lax.broadcasted_iota(jnp.int32, (1, NPAD), 1)
    row_valid = col < N
    probs_ref[...] = jnp.where(row_valid, e / denom, 0.0)

    src_h = rois_ref[2:3, :] - rois_ref[0:1, :]
    src_w = rois_ref[3:4, :] - rois_ref[1:2, :]
    ctr_y = rois_ref[0:1, :] + 0.5 * src_h
    ctr_x = rois_ref[1:2, :] + 0.5 * src_w

    dy = loc_ref[0] * 0.1
    dx = loc_ref[1] * 0.1
    dh = loc_ref[2] * 0.2
    dw = loc_ref[3] * 0.2
    h = jnp.exp(dh) * src_h
    w = jnp.exp(dw) * src_w
    cy = dy * src_h + ctr_y
    cx = dx * src_w + ctr_x
    boxes_ref[0] = jnp.clip(cy - 0.5 * h, 0.0, IMG_H)
    boxes_ref[1] = jnp.clip(cx - 0.5 * w, 0.0, IMG_W)
    boxes_ref[2] = jnp.clip(cy + 0.5 * h, 0.0, IMG_H)
    boxes_ref[3] = jnp.clip(cx + 0.5 * w, 0.0, IMG_W)


def _nms_body(probs_hbm, boxes_hbm, ob_hbm, os_hbm, ol_hbm,
              score_v, boxes_v, cm_v, cm2_v, sel_v, ob_v, os_v, ol_v):
    cidx = lax.axis_index("c")
    sidx = lax.axis_index("s")
    wid = sidx * 2 + cidx

    @pl.when(wid < NFG)
    def _work():
        cls = wid + 1
        pltpu.sync_copy(probs_hbm.at[cls], score_v)
        pltpu.sync_copy(boxes_hbm.at[0, cls], boxes_v.at[0])
        pltpu.sync_copy(boxes_hbm.at[1, cls], boxes_v.at[1])
        pltpu.sync_copy(boxes_hbm.at[2, cls], boxes_v.at[2])
        pltpu.sync_copy(boxes_hbm.at[3, cls], boxes_v.at[3])
        lane = lax.iota(jnp.int32, 16)
        zf16 = jnp.zeros((16,), jnp.float32)
        negv = jnp.full((16,), NEG, jnp.float32)

        def zinit_ob(i, _):
            ob_v[pl.ds(i * 16, 16)] = zf16
            return 0

        lax.fori_loop(0, (KPAD * 4) // 16, zinit_ob, 0)

        def zinit_os(i, _):
            os_v[pl.ds(i * 16, 16)] = zf16
            sel_v[0, pl.ds(i * 16, 16)] = zf16
            sel_v[1, pl.ds(i * 16, 16)] = zf16
            sel_v[2, pl.ds(i * 16, 16)] = zf16
            sel_v[3, pl.ds(i * 16, 16)] = zf16
            sel_v[4, pl.ds(i * 16, 16)] = zf16
            return 0

        lax.fori_loop(0, KPAD // 16, zinit_os, 0)

        def cmneg(i, _):
            cm_v[pl.ds(i * 16, 16)] = negv
            return 0

        lax.fori_loop(0, (NCHUNK + 16) // 16, cmneg, 0)
        cm2_v[pl.ds(0, 16)] = negv
        cm2_v[pl.ds(16, 16)] = negv

        def cmb(i, _):
            for k in range(4):
                j = i * 4 + k
                p = score_v[pl.ds(j * 16, 16)]
                s = jnp.where(p > SCORE_THRESH, p, NEG)
                score_v[pl.ds(j * 16, 16)] = s
                plsc.store_scatter(cm_v, [jnp.full((16,), j, jnp.int32)],
                                   jnp.full((16,), jnp.max(s), jnp.float32),
                                   mask=lane == 0)
            return 0

        lax.fori_loop(0, NCHUNK // 4, cmb, 0)

        def cm2b(g, _):
            cmg = cm_v[pl.ds(g * 16, 16)]
            plsc.store_scatter(cm2_v, [jnp.full((16,), g, jnp.int32)],
                               jnp.full((16,), jnp.max(cmg), jnp.float32),
                               mask=lane == 0)
            return 0

        lax.fori_loop(0, NCHUNK // 16, cm2b, 0)

        m0 = jnp.max(jnp.maximum(cm2_v[pl.ds(0, 16)], cm2_v[pl.ds(16, 16)]))

        def cond(carry):
            ns, m = carry
            return (ns < K) & (m > 0.0)

        def body(carry):
            ns, m = carry
            c0 = cm2_v[pl.ds(0, 16)]
            c1 = cm2_v[pl.ds(16, 16)]
            f0 = plsc.all_reduce_ffs(c0 >= m)[0]
            f1 = plsc.all_reduce_ffs(c1 >= m)[0]
            g = jnp.where(f0 < 16, f0, 16 + f1)
            cmg = cm_v[pl.ds(g * 16, 16)]
            jin = plsc.all_reduce_ffs(cmg >= m)[0]
            j = g * 16 + jin
            s = score_v[pl.ds(j * 16, 16)]
            lsel = plsc.all_reduce_ffs(s >= m)[0]
            idx = j * 16 + lsel
            s2 = jnp.where(lane == lsel, NEG, s)
            score_v[pl.ds(j * 16, 16)] = s2
            cmj = jnp.max(s2)
            cmg2 = jnp.where(lane == jin, cmj, cmg)
            cm_v[pl.ds(g * 16, 16)] = cmg2
            maxg = jnp.max(cmg2)
            plsc.store_scatter(cm2_v, [jnp.full((16,), g, jnp.int32)],
                               jnp.full((16,), maxg, jnp.float32),
                               mask=lane == 0)
            c0n = jnp.where(lane == g, maxg, c0)
            c1n = jnp.where(lane + 16 == g, maxg, c1)
            m2 = jnp.max(jnp.maximum(c0n, c1n))

            idxv = jnp.full((16,), idx, jnp.int32)
            by1 = plsc.load_gather(boxes_v, [jnp.zeros((16,), jnp.int32), idxv])
            bx1 = plsc.load_gather(boxes_v, [jnp.full((16,), 1, jnp.int32), idxv])
            by2 = plsc.load_gather(boxes_v, [jnp.full((16,), 2, jnp.int32), idxv])
            bx2 = plsc.load_gather(boxes_v, [jnp.full((16,), 3, jnp.int32), idxv])
            ca = jnp.maximum(by2 - by1, 0.0) * jnp.maximum(bx2 - bx1, 0.0)

            def tchk(tb, mi):
                sy1 = sel_v[0, pl.ds(tb, 16)]
                sx1 = sel_v[1, pl.ds(tb, 16)]
                sy2 = sel_v[2, pl.ds(tb, 16)]
                sx2 = sel_v[3, pl.ds(tb, 16)]
                sa = sel_v[4, pl.ds(tb, 16)]
                tl_y = jnp.maximum(sy1, by1)
                tl_x = jnp.maximum(sx1, bx1)
                br_y = jnp.minimum(sy2, by2)
                br_x = jnp.minimum(sx2, bx2)
                wh_y = jnp.maximum(br_y - tl_y, 0.0)
                wh_x = jnp.maximum(br_x - tl_x, 0.0)
                inter = wh_y * wh_x
                iou = inter / (sa + ca - inter + 1e-9)
                return jnp.maximum(mi, iou)

            maxiou_v = plsc.parallel_loop(
                0, ((ns + 15) // 16) * 16, 16, unroll=2, carry=zf16)(tchk)
            keep = jnp.max(maxiou_v) <= NMS_THRESH

            @pl.when(keep)
            def _acc():
                nsv = jnp.full((16,), ns, jnp.int32)
                lane0 = lane == 0
                zi0 = jnp.zeros((16,), jnp.int32)
                plsc.store_scatter(sel_v, [zi0, nsv], by1, mask=lane0)
                plsc.store_scatter(sel_v, [zi0 + 1, nsv], bx1, mask=lane0)
                plsc.store_scatter(sel_v, [zi0 + 2, nsv], by2, mask=lane0)
                plsc.store_scatter(sel_v, [zi0 + 3, nsv], bx2, mask=lane0)
                plsc.store_scatter(sel_v, [zi0 + 4, nsv], ca, mask=lane0)
                boxvec = jnp.where(lane == 0, by1,
                                   jnp.where(lane == 1, bx1,
                                             jnp.where(lane == 2, by2, bx2)))
                plsc.store_scatter(ob_v, [ns * 4 + lane], boxvec,
                                   mask=lane < 4)
                plsc.store_scatter(os_v, [nsv],
                                   jnp.full((16,), m, jnp.float32),
                                   mask=lane0)

            ns2 = jnp.where(keep, ns + 1, ns)
            return ns2, m2

        kfin, _ = lax.while_loop(cond, body, (jnp.int32(0), m0))

        def lfill(i, _):
            base = i * 16
            ol_v[pl.ds(base, 16)] = jnp.where(base + lane < kfin, wid, -1)
            return 0

        lax.fori_loop(0, KPAD // 16, lfill, 0)

        pltpu.sync_copy(ob_v, ob_hbm.at[wid])
        pltpu.sync_copy(os_v, os_hbm.at[wid])
        pltpu.sync_copy(ol_v, ol_hbm.at[wid])


@jax.jit
def kernel(roi_cls_loc, roi_scores, rois):
    pad = NPAD - N
    scores_t = jnp.pad(roi_scores, ((0, pad), (0, 0))).T
    loc_t = jnp.transpose(
        jnp.pad(roi_cls_loc.reshape(N, NCLS, 4), ((0, pad), (0, 0), (0, 0))),
        (2, 1, 0))
    rois_t = jnp.pad(rois, ((0, pad), (0, 0))).T

    probs, boxes = pl.pallas_call(
        _prep_body,
        out_shape=[
            jax.ShapeDtypeStruct((NCLS, NPAD), jnp.float32),
            jax.ShapeDtypeStruct((4, NCLS, NPAD), jnp.float32),
        ],
    )(scores_t, loc_t, rois_t)

    nms = pl.kernel(
        _nms_body,
        out_type=[
            jax.ShapeDtypeStruct((NFG, KPAD * 4), jnp.float32),
            jax.ShapeDtypeStruct((NFG, KPAD), jnp.float32),
            jax.ShapeDtypeStruct((NFG, KPAD), jnp.int32),
        ],
        mesh=plsc.VectorSubcoreMesh(core_axis_name="c", subcore_axis_name="s"),
        compiler_params=pltpu.CompilerParams(needs_layout_passes=False),
        scratch_types=[
            pltpu.VMEM((NPAD,), jnp.float32),
            pltpu.VMEM((4, NPAD), jnp.float32),
            pltpu.VMEM((NCHUNK + 16,), jnp.float32),
            pltpu.VMEM((32,), jnp.float32),
            pltpu.VMEM((5, KPAD), jnp.float32),
            pltpu.VMEM((KPAD * 4,), jnp.float32),
            pltpu.VMEM((KPAD,), jnp.float32),
            pltpu.VMEM((KPAD,), jnp.int32),
        ],
    )
    ob, os_, ol = nms(probs, boxes)

    out_boxes = ob.reshape(NFG, KPAD, 4)[:, :K, :].reshape(-1, 4)
    out_scores = os_[:, :K].reshape(-1)
    out_labels = ol[:, :K].reshape(-1)
    return out_boxes, out_labels, out_scores


# speedup vs baseline: 74.4542x; 1.0522x over previous
import functools

import jax
import jax.numpy as jnp
from jax import lax
from jax.experimental import pallas as pl
from jax.experimental.pallas import tpu as pltpu, tpu_sc as plsc

N = 5000
NPAD = 5120
NCHUNK = NPAD // 16
NCLS = 21
NFG = NCLS - 1
K = 100
KPAD = 128
NMS_THRESH = 0.3
SCORE_THRESH = 0.05
IMG_H = 600.0
IMG_W = 800.0
NEG = -1.0


def _prep_body(scores_ref, loc_ref, rois_ref, probs_ref, boxes_ref):
    scores = scores_ref[...]
    mx = jnp.max(scores, axis=0, keepdims=True)
    e = jnp.exp(scores - mx)
    denom = jnp.sum(e, axis=0, keepdims=True)
    col = lax.broadcasted_iota(jnp.int32, (1, NPAD), 1)
    row_valid = col < N
    probs_ref[...] = jnp.where(row_valid, e / denom, 0.0)

    src_h = rois_ref[2:3, :] - rois_ref[0:1, :]
    src_w = rois_ref[3:4, :] - rois_ref[1:2, :]
    ctr_y = rois_ref[0:1, :] + 0.5 * src_h
    ctr_x = rois_ref[1:2, :] + 0.5 * src_w

    dy = loc_ref[0] * 0.1
    dx = loc_ref[1] * 0.1
    dh = loc_ref[2] * 0.2
    dw = loc_ref[3] * 0.2
    h = jnp.exp(dh) * src_h
    w = jnp.exp(dw) * src_w
    cy = dy * src_h + ctr_y
    cx = dx * src_w + ctr_x
    boxes_ref[0] = jnp.clip(cy - 0.5 * h, 0.0, IMG_H)
    boxes_ref[1] = jnp.clip(cx - 0.5 * w, 0.0, IMG_W)
    boxes_ref[2] = jnp.clip(cy + 0.5 * h, 0.0, IMG_H)
    boxes_ref[3] = jnp.clip(cx + 0.5 * w, 0.0, IMG_W)


def _nms_body(probs_hbm, boxes_hbm, ob_hbm, os_hbm, ol_hbm,
              score_v, boxes_v, cm_v, cm2_v, sel_v, ob_v, os_v, ol_v, dsem):
    cidx = lax.axis_index("c")
    sidx = lax.axis_index("s")
    wid = sidx * 2 + cidx

    @pl.when(wid < NFG)
    def _work():
        cls = wid + 1
        h1 = pltpu.async_copy(probs_hbm.at[cls], score_v, dsem)
        h2 = pltpu.async_copy(boxes_hbm.at[0, cls], boxes_v.at[0], dsem)
        h3 = pltpu.async_copy(boxes_hbm.at[1, cls], boxes_v.at[1], dsem)
        h4 = pltpu.async_copy(boxes_hbm.at[2, cls], boxes_v.at[2], dsem)
        h5 = pltpu.async_copy(boxes_hbm.at[3, cls], boxes_v.at[3], dsem)
        lane = lax.iota(jnp.int32, 16)
        zf16 = jnp.zeros((16,), jnp.float32)
        negv = jnp.full((16,), NEG, jnp.float32)

        @plsc.parallel_loop(0, (KPAD * 4) // 16, unroll=4)
        def _zob(i):
            ob_v[pl.ds(i * 16, 16)] = zf16

        @plsc.parallel_loop(0, KPAD // 16, unroll=2)
        def _zos(i):
            os_v[pl.ds(i * 16, 16)] = zf16
            sel_v[0, pl.ds(i * 16, 16)] = zf16
            sel_v[1, pl.ds(i * 16, 16)] = zf16
            sel_v[2, pl.ds(i * 16, 16)] = zf16
            sel_v[3, pl.ds(i * 16, 16)] = zf16
            sel_v[4, pl.ds(i * 16, 16)] = zf16

        @plsc.parallel_loop(0, (NCHUNK + 16) // 16, unroll=4)
        def _zcm(i):
            cm_v[pl.ds(i * 16, 16)] = negv

        cm2_v[pl.ds(0, 16)] = negv
        cm2_v[pl.ds(16, 16)] = negv
        h1.wait()
        h2.wait()
        h3.wait()
        h4.wait()
        h5.wait()

        def cmb(i, _):
            for k in range(4):
                j = i * 4 + k
                p = score_v[pl.ds(j * 16, 16)]
                s = jnp.where(p > SCORE_THRESH, p, NEG)
                score_v[pl.ds(j * 16, 16)] = s
                plsc.store_scatter(cm_v, [jnp.full((16,), j, jnp.int32)],
                                   jnp.full((16,), jnp.max(s), jnp.float32),
                                   mask=lane == 0)
            return 0

        lax.fori_loop(0, NCHUNK // 4, cmb, 0)

        def cm2b(g, _):
            cmg = cm_v[pl.ds(g * 16, 16)]
            plsc.store_scatter(cm2_v, [jnp.full((16,), g, jnp.int32)],
                               jnp.full((16,), jnp.max(cmg), jnp.float32),
                               mask=lane == 0)
            return 0

        lax.fori_loop(0, NCHUNK // 16, cm2b, 0)

        m0 = jnp.max(jnp.maximum(cm2_v[pl.ds(0, 16)], cm2_v[pl.ds(16, 16)]))

        def cond(carry):
            ns, m = carry
            return (ns < K) & (m > 0.0)

        def body(carry):
            ns, m = carry
            c0 = cm2_v[pl.ds(0, 16)]
            c1 = cm2_v[pl.ds(16, 16)]
            f0 = plsc.all_reduce_ffs(c0 >= m)[0]
            f1 = plsc.all_reduce_ffs(c1 >= m)[0]
            g = jnp.where(f0 < 16, f0, 16 + f1)
            cmg = cm_v[pl.ds(g * 16, 16)]
            jin = plsc.all_reduce_ffs(cmg >= m)[0]
            j = g * 16 + jin
            s = score_v[pl.ds(j * 16, 16)]
            lsel = plsc.all_reduce_ffs(s >= m)[0]
            idx = j * 16 + lsel
            s2 = jnp.where(lane == lsel, NEG, s)
            score_v[pl.ds(j * 16, 16)] = s2
            cmj = jnp.max(s2)
            cmg2 = jnp.where(lane == jin, cmj, cmg)
            cm_v[pl.ds(g * 16, 16)] = cmg2
            maxg = jnp.max(cmg2)
            plsc.store_scatter(cm2_v, [jnp.full((16,), g, jnp.int32)],
                               jnp.full((16,), maxg, jnp.float32),
                               mask=lane == 0)
            c0n = jnp.where(lane == g, maxg, c0)
            c1n = jnp.where(lane + 16 == g, maxg, c1)
            m2 = jnp.max(jnp.maximum(c0n, c1n))

            idxv = jnp.full((16,), idx, jnp.int32)
            by1 = plsc.load_gather(boxes_v, [jnp.zeros((16,), jnp.int32), idxv])
            bx1 = plsc.load_gather(boxes_v, [jnp.full((16,), 1, jnp.int32), idxv])
            by2 = plsc.load_gather(boxes_v, [jnp.full((16,), 2, jnp.int32), idxv])
            bx2 = plsc.load_gather(boxes_v, [jnp.full((16,), 3, jnp.int32), idxv])
            ca = jnp.maximum(by2 - by1, 0.0) * jnp.maximum(bx2 - bx1, 0.0)

            def tchk(tb, mi):
                sy1 = sel_v[0, pl.ds(tb, 16)]
                sx1 = sel_v[1, pl.ds(tb, 16)]
                sy2 = sel_v[2, pl.ds(tb, 16)]
                sx2 = sel_v[3, pl.ds(tb, 16)]
                sa = sel_v[4, pl.ds(tb, 16)]
                tl_y = jnp.maximum(sy1, by1)
                tl_x = jnp.maximum(sx1, bx1)
                br_y = jnp.minimum(sy2, by2)
                br_x = jnp.minimum(sx2, bx2)
                wh_y = jnp.maximum(br_y - tl_y, 0.0)
                wh_x = jnp.maximum(br_x - tl_x, 0.0)
                inter = wh_y * wh_x
                iou = inter / (sa + ca - inter + 1e-9)
                return jnp.maximum(mi, iou)

            maxiou_v = plsc.parallel_loop(
                0, ((ns + 15) // 16) * 16, 16, unroll=2, carry=zf16)(tchk)
            keep = jnp.max(maxiou_v) <= NMS_THRESH

            @pl.when(keep)
            def _acc():
                nsv = jnp.full((16,), ns, jnp.int32)
                lane0 = lane == 0
                zi0 = jnp.zeros((16,), jnp.int32)
                plsc.store_scatter(sel_v, [zi0, nsv], by1, mask=lane0)
                plsc.store_scatter(sel_v, [zi0 + 1, nsv], bx1, mask=lane0)
                plsc.store_scatter(sel_v, [zi0 + 2, nsv], by2, mask=lane0)
                plsc.store_scatter(sel_v, [zi0 + 3, nsv], bx2, mask=lane0)
                plsc.store_scatter(sel_v, [zi0 + 4, nsv], ca, mask=lane0)
                boxvec = jnp.where(lane == 0, by1,
                                   jnp.where(lane == 1, bx1,
                                             jnp.where(lane == 2, by2, bx2)))
                plsc.store_scatter(ob_v, [ns * 4 + lane], boxvec,
                                   mask=lane < 4)
                plsc.store_scatter(os_v, [nsv],
                                   jnp.full((16,), m, jnp.float32),
                                   mask=lane0)

            ns2 = jnp.where(keep, ns + 1, ns)
            return ns2, m2

        kfin, _ = lax.while_loop(cond, body, (jnp.int32(0), m0))

        def lfill(i, _):
            base = i * 16
            ol_v[pl.ds(base, 16)] = jnp.where(base + lane < kfin, wid, -1)
            return 0

        lax.fori_loop(0, KPAD // 16, lfill, 0)

        pltpu.sync_copy(ob_v, ob_hbm.at[wid])
        pltpu.sync_copy(os_v, os_hbm.at[wid])
        pltpu.sync_copy(ol_v, ol_hbm.at[wid])


@jax.jit
def kernel(roi_cls_loc, roi_scores, rois):
    pad = NPAD - N
    scores_t = jnp.pad(roi_scores, ((0, pad), (0, 0))).T
    loc_t = jnp.transpose(
        jnp.pad(roi_cls_loc.reshape(N, NCLS, 4), ((0, pad), (0, 0), (0, 0))),
        (2, 1, 0))
    rois_t = jnp.pad(rois, ((0, pad), (0, 0))).T

    probs, boxes = pl.pallas_call(
        _prep_body,
        out_shape=[
            jax.ShapeDtypeStruct((NCLS, NPAD), jnp.float32),
            jax.ShapeDtypeStruct((4, NCLS, NPAD), jnp.float32),
        ],
    )(scores_t, loc_t, rois_t)

    nms = pl.kernel(
        _nms_body,
        out_type=[
            jax.ShapeDtypeStruct((NFG, KPAD * 4), jnp.float32),
            jax.ShapeDtypeStruct((NFG, KPAD), jnp.float32),
            jax.ShapeDtypeStruct((NFG, KPAD), jnp.int32),
        ],
        mesh=plsc.VectorSubcoreMesh(core_axis_name="c", subcore_axis_name="s"),
        compiler_params=pltpu.CompilerParams(needs_layout_passes=False),
        scratch_types=[
            pltpu.VMEM((NPAD,), jnp.float32),
            pltpu.VMEM((4, NPAD), jnp.float32),
            pltpu.VMEM((NCHUNK + 16,), jnp.float32),
            pltpu.VMEM((32,), jnp.float32),
            pltpu.VMEM((5, KPAD), jnp.float32),
            pltpu.VMEM((KPAD * 4,), jnp.float32),
            pltpu.VMEM((KPAD,), jnp.float32),
            pltpu.VMEM((KPAD,), jnp.int32),
            pltpu.SemaphoreType.DMA,
        ],
    )
    ob, os_, ol = nms(probs, boxes)

    out_boxes = ob.reshape(NFG, KPAD, 4)[:, :K, :].reshape(-1, 4)
    out_scores = os_[:, :K].reshape(-1)
    out_labels = ol[:, :K].reshape(-1)
    return out_boxes, out_labels, out_scores
